# baseline (speedup 1.0000x reference)
import jax
import jax.numpy as jnp
from jax import lax
from jax.experimental import pallas as pl
from jax.experimental.pallas import tpu as pltpu

_B, _S, _H, _E, _K = 1, 2048, 1024, 16, 2
_CAP = 384
_TS = 256
_GRID = _S // _TS


def _router_body(x_ref, w1_ref, b1_ref, w2_ref, b2_ref,
                 disp_ref, probs_ref, comb0_ref, aux_ref, acc_ref):
    x = x_ref[0]
    h = jnp.dot(x, w1_ref[...], preferred_element_type=jnp.float32)
    h = jnp.maximum(h + b1_ref[...], 0.0)
    logits = jnp.dot(h, w2_ref[...], preferred_element_type=jnp.float32)
    logits = logits + b2_ref[...]

    m = jnp.max(logits, axis=1, keepdims=True)
    ex = jnp.exp(logits - m)
    p = ex / jnp.sum(ex, axis=1, keepdims=True)
    probs_ref[0] = p.T

    idx = lax.broadcasted_iota(jnp.int32, (_TS, _E), 1)
    m1 = jnp.max(p, axis=1, keepdims=True)
    i1 = jnp.min(jnp.where(p == m1, idx, _E), axis=1, keepdims=True)
    mask1 = idx == i1
    pm = jnp.where(mask1, -1.0, p)
    m2 = jnp.max(pm, axis=1, keepdims=True)
    i2 = jnp.min(jnp.where(pm == m2, idx, _E), axis=1, keepdims=True)
    mask2 = idx == i2
    denom = m1 + m2
    comb0_ref[...] = (jnp.where(mask1, m1, 0.0)
                      + jnp.where(mask2, m2, 0.0)) / denom
    disp0 = (mask1 | mask2).astype(jnp.float32)

    cap0 = lax.broadcasted_iota(jnp.int32, (_TS, _E, _CAP), 2) == 0
    disp_ref[0] = jnp.where(cap0, disp0[:, :, None], 0.0)

    step = pl.program_id(0)
    psum = jnp.sum(p, axis=0, keepdims=True)

    @pl.when(step == 0)
    def _():
        acc_ref[...] = psum

    @pl.when(step != 0)
    def _():
        acc_ref[...] = acc_ref[...] + psum

    @pl.when(step == _GRID - 1)
    def _():
        rp = acc_ref[...] / (_B * _S)
        aux_ref[0, 0] = jnp.sum(rp * jnp.log(rp * _E + 1e-09))


_router_call = pl.pallas_call(
    _router_body,
    grid=(_GRID,),
    in_specs=[
        pl.BlockSpec((1, _TS, _H), lambda i: (0, i, 0)),
        pl.BlockSpec((_H, _H), lambda i: (0, 0)),
        pl.BlockSpec((1, _H), lambda i: (0, 0)),
        pl.BlockSpec((_H, _E), lambda i: (0, 0)),
        pl.BlockSpec((1, _E), lambda i: (0, 0)),
    ],
    out_specs=[
        pl.BlockSpec((1, _TS, _E, _CAP), lambda i: (0, i, 0, 0)),
        pl.BlockSpec((1, _E, _TS), lambda i: (0, 0, i)),
        pl.BlockSpec((_TS, _E), lambda i: (i, 0)),
        pl.BlockSpec((1, 1), lambda i: (0, 0), memory_space=pltpu.SMEM),
    ],
    out_shape=[
        jax.ShapeDtypeStruct((_B, _S, _E, _CAP), jnp.float32),
        jax.ShapeDtypeStruct((_B, _E, _S), jnp.float32),
        jax.ShapeDtypeStruct((_S, _E), jnp.float32),
        jax.ShapeDtypeStruct((1, 1), jnp.float32),
    ],
    scratch_shapes=[pltpu.VMEM((1, _E), jnp.float32)],
)


def _comb_body(c0_ref, out_ref):
    cap0 = lax.broadcasted_iota(jnp.int32, (_TS, _E, _CAP), 2) == 0
    out_ref[0] = jnp.where(cap0, c0_ref[...][:, :, None], 0.0)


_comb_call = pl.pallas_call(
    _comb_body,
    grid=(_GRID,),
    in_specs=[pl.BlockSpec((_TS, _E), lambda i: (i, 0))],
    out_specs=[pl.BlockSpec((1, _TS, _E, _CAP), lambda i: (0, i, 0, 0))],
    out_shape=[jax.ShapeDtypeStruct((_B, _S, _E, _CAP), jnp.float32)],
)


def kernel(hidden_states, W1, b1, W2, b2):
    disp, probs_t, comb0, aux = _router_call(
        hidden_states, W1, b1.reshape(1, _H), W2, b2.reshape(1, _E))
    comb, = _comb_call(comb0)
    return (disp, comb, jnp.transpose(probs_t, (0, 2, 1)), aux[0, 0])


# speedup vs baseline: 1.0989x; 1.0989x over previous
import jax
import jax.numpy as jnp
from jax import lax
from jax.experimental import pallas as pl
from jax.experimental.pallas import tpu as pltpu

_B, _S, _H, _E, _K = 1, 2048, 1024, 16, 2
_CAP = 384
_TS = 256
_GRID = _S // _TS


def _router_body(x_hbm, w1_ref, b1_ref, w2_ref, b2_ref,
                 disp_ref, comb_ref, probs_ref, aux_ref,
                 acc_ref, xbuf_ref, sems):
    i = pl.program_id(0)
    slot = lax.rem(i, 2)
    nxt = lax.rem(i + 1, 2)

    @pl.when(i == 0)
    def _():
        pltpu.make_async_copy(x_hbm.at[0, pl.ds(0, _TS), :],
                              xbuf_ref.at[0], sems.at[0]).start()

    @pl.when(i + 1 < _GRID)
    def _():
        pltpu.make_async_copy(x_hbm.at[0, pl.ds((i + 1) * _TS, _TS), :],
                              xbuf_ref.at[nxt], sems.at[nxt]).start()

    pltpu.make_async_copy(x_hbm.at[0, pl.ds(i * _TS, _TS), :],
                          xbuf_ref.at[slot], sems.at[slot]).wait()

    x = xbuf_ref[slot]
    h = jnp.dot(x, w1_ref[...], preferred_element_type=jnp.float32)
    h = jnp.maximum(h + b1_ref[...], 0.0)
    logits = jnp.dot(h, w2_ref[...], preferred_element_type=jnp.float32)
    logits = logits + b2_ref[...]

    m = jnp.max(logits, axis=1, keepdims=True)
    ex = jnp.exp(logits - m)
    p = ex / jnp.sum(ex, axis=1, keepdims=True)
    probs_ref[0] = p.T

    idx = lax.broadcasted_iota(jnp.int32, (_TS, _E), 1)
    m1 = jnp.max(p, axis=1, keepdims=True)
    i1 = jnp.min(jnp.where(p == m1, idx, _E), axis=1, keepdims=True)
    mask1 = idx == i1
    pm = jnp.where(mask1, -1.0, p)
    m2 = jnp.max(pm, axis=1, keepdims=True)
    i2 = jnp.min(jnp.where(pm == m2, idx, _E), axis=1, keepdims=True)
    mask2 = idx == i2
    denom = m1 + m2
    comb0 = (jnp.where(mask1, m1, 0.0) + jnp.where(mask2, m2, 0.0)) / denom
    disp0 = (mask1 | mask2).astype(jnp.float32)

    cap0 = lax.broadcasted_iota(jnp.int32, (_TS, _E, _CAP), 2) == 0
    disp_ref[0] = jnp.where(cap0, disp0[:, :, None], 0.0)
    comb_ref[0] = jnp.where(cap0, comb0[:, :, None], 0.0)

    psum = jnp.sum(p, axis=0, keepdims=True)

    @pl.when(i == 0)
    def _():
        acc_ref[...] = psum

    @pl.when(i != 0)
    def _():
        acc_ref[...] = acc_ref[...] + psum

    @pl.when(i == _GRID - 1)
    def _():
        rp = acc_ref[...] / (_B * _S)
        aux_ref[0, 0] = jnp.sum(rp * jnp.log(rp * _E + 1e-09))


_call = pl.pallas_call(
    _router_body,
    grid=(_GRID,),
    in_specs=[
        pl.BlockSpec(memory_space=pl.ANY),
        pl.BlockSpec((_H, _H), lambda i: (0, 0)),
        pl.BlockSpec((1, _H), lambda i: (0, 0)),
        pl.BlockSpec((_H, _E), lambda i: (0, 0)),
        pl.BlockSpec((1, _E), lambda i: (0, 0)),
    ],
    out_specs=[
        pl.BlockSpec((1, _TS, _E, _CAP), lambda i: (0, i, 0, 0)),
        pl.BlockSpec((1, _TS, _E, _CAP), lambda i: (0, i, 0, 0)),
        pl.BlockSpec((1, _E, _TS), lambda i: (0, 0, i)),
        pl.BlockSpec((1, 1), lambda i: (0, 0), memory_space=pltpu.SMEM),
    ],
    out_shape=[
        jax.ShapeDtypeStruct((_B, _S, _E, _CAP), jnp.float32),
        jax.ShapeDtypeStruct((_B, _S, _E, _CAP), jnp.float32),
        jax.ShapeDtypeStruct((_B, _E, _S), jnp.float32),
        jax.ShapeDtypeStruct((1, 1), jnp.float32),
    ],
    scratch_shapes=[
        pltpu.VMEM((1, _E), jnp.float32),
        pltpu.VMEM((2, _TS, _H), jnp.float32),
        pltpu.SemaphoreType.DMA((2,)),
    ],
)


def kernel(hidden_states, W1, b1, W2, b2):
    disp, comb, probs_t, aux = _call(
        hidden_states, W1, b1.reshape(1, _H), W2, b2.reshape(1, _E))
    return (disp, comb, jnp.transpose(probs_t, (0, 2, 1)), aux[0, 0])


# speedup vs baseline: 1.1195x; 1.0188x over previous
import jax
import jax.numpy as jnp
from jax import lax
from jax.experimental import pallas as pl
from jax.experimental.pallas import tpu as pltpu

_B, _S, _H, _E, _K = 1, 2048, 1024, 16, 2
_CAP = 384
_TS = 256
_GRID = _S // _TS


def _router_body(x_ref, w1_ref, b1_ref, w2_ref, b2_ref,
                 disp_ref, comb_ref, probs_ref, aux_ref, acc_ref):
    x = x_ref[0]
    h = jnp.dot(x, w1_ref[...], preferred_element_type=jnp.float32)
    h = jnp.maximum(h + b1_ref[...], 0.0)
    logits = jnp.dot(h, w2_ref[...], preferred_element_type=jnp.float32)
    logits = logits + b2_ref[...]

    m = jnp.max(logits, axis=1, keepdims=True)
    ex = jnp.exp(logits - m)
    p = ex / jnp.sum(ex, axis=1, keepdims=True)
    probs_ref[0] = p.T

    idx = lax.broadcasted_iota(jnp.int32, (_TS, _E), 1)
    m1 = jnp.max(p, axis=1, keepdims=True)
    i1 = jnp.min(jnp.where(p == m1, idx, _E), axis=1, keepdims=True)
    mask1 = idx == i1
    pm = jnp.where(mask1, -1.0, p)
    m2 = jnp.max(pm, axis=1, keepdims=True)
    i2 = jnp.min(jnp.where(pm == m2, idx, _E), axis=1, keepdims=True)
    mask2 = idx == i2
    denom = m1 + m2
    comb0 = (jnp.where(mask1, m1, 0.0) + jnp.where(mask2, m2, 0.0)) / denom
    disp0 = (mask1 | mask2).astype(jnp.float32)

    cap0 = lax.broadcasted_iota(jnp.int32, (_TS, _E, _CAP), 2) == 0
    disp_ref[0] = jnp.where(cap0, disp0[:, :, None], 0.0)
    comb_ref[0] = jnp.where(cap0, comb0[:, :, None], 0.0)

    step = pl.program_id(0)
    psum = jnp.sum(p, axis=0, keepdims=True)

    @pl.when(step == 0)
    def _():
        acc_ref[...] = psum

    @pl.when(step != 0)
    def _():
        acc_ref[...] = acc_ref[...] + psum

    @pl.when(step == _GRID - 1)
    def _():
        rp = acc_ref[...] / (_B * _S)
        aux_ref[0, 0] = jnp.sum(rp * jnp.log(rp * _E + 1e-09))


_call = pl.pallas_call(
    _router_body,
    grid=(_GRID,),
    in_specs=[
        pl.BlockSpec((1, _TS, _H), lambda i: (0, i, 0)),
        pl.BlockSpec((_H, _H), lambda i: (0, 0)),
        pl.BlockSpec((1, _H), lambda i: (0, 0)),
        pl.BlockSpec((_H, _E), lambda i: (0, 0)),
        pl.BlockSpec((1, _E), lambda i: (0, 0)),
    ],
    out_specs=[
        pl.BlockSpec((1, _TS, _E, _CAP), lambda i: (0, i, 0, 0)),
        pl.BlockSpec((1, _TS, _E, _CAP), lambda i: (0, i, 0, 0)),
        pl.BlockSpec((1, _E, _TS), lambda i: (0, 0, i)),
        pl.BlockSpec((1, 1), lambda i: (0, 0), memory_space=pltpu.SMEM),
    ],
    out_shape=[
        jax.ShapeDtypeStruct((_B, _S, _E, _CAP), jnp.float32),
        jax.ShapeDtypeStruct((_B, _S, _E, _CAP), jnp.float32),
        jax.ShapeDtypeStruct((_B, _E, _S), jnp.float32),
        jax.ShapeDtypeStruct((1, 1), jnp.float32),
    ],
    scratch_shapes=[pltpu.VMEM((1, _E), jnp.float32)],
)


def kernel(hidden_states, W1, b1, W2, b2):
    disp, comb, probs_t, aux = _call(
        hidden_states, W1, b1.reshape(1, _H), W2, b2.reshape(1, _E))
    return (disp, comb, jnp.transpose(probs_t, (0, 2, 1)), aux[0, 0])


# speedup vs baseline: 1.1215x; 1.0017x over previous
import jax
import jax.numpy as jnp
from jax import lax
from jax.experimental import pallas as pl
from jax.experimental.pallas import tpu as pltpu

_B, _S, _H, _E, _K = 1, 2048, 1024, 16, 2
_CAP = 384
_TS = 256
_GRID = _S // _TS


def _router_body(x_ref, w1_ref, b1_ref, w2_ref, b2_ref,
                 disp_ref, comb_ref, probs_ref, aux_ref, acc_ref):
    x = x_ref[0]
    h = jnp.dot(x, w1_ref[...], preferred_element_type=jnp.float32)
    h = jnp.maximum(h + b1_ref[...], 0.0)
    logits = jnp.dot(h, w2_ref[...], preferred_element_type=jnp.float32)
    logits = logits + b2_ref[...]

    m = jnp.max(logits, axis=1, keepdims=True)
    ex = jnp.exp(logits - m)
    p = ex / jnp.sum(ex, axis=1, keepdims=True)
    probs_ref[0] = p.T

    idx = lax.broadcasted_iota(jnp.int32, (_TS, _E), 1)
    m1 = jnp.max(p, axis=1, keepdims=True)
    i1 = jnp.min(jnp.where(p == m1, idx, _E), axis=1, keepdims=True)
    mask1 = idx == i1
    pm = jnp.where(mask1, -1.0, p)
    m2 = jnp.max(pm, axis=1, keepdims=True)
    i2 = jnp.min(jnp.where(pm == m2, idx, _E), axis=1, keepdims=True)
    mask2 = idx == i2
    denom = m1 + m2
    comb0 = (jnp.where(mask1, m1, 0.0) + jnp.where(mask2, m2, 0.0)) / denom
    disp0 = (mask1 | mask2).astype(jnp.float32)

    cap0 = lax.broadcasted_iota(jnp.int32, (_TS, _E, 128), 2) == 0
    z = jnp.zeros((_TS, _E, _CAP - 128), jnp.float32)
    disp_ref[0, :, :, 0:128] = jnp.where(cap0, disp0[:, :, None], 0.0)
    disp_ref[0, :, :, 128:_CAP] = z
    comb_ref[0, :, :, 0:128] = jnp.where(cap0, comb0[:, :, None], 0.0)
    comb_ref[0, :, :, 128:_CAP] = z

    step = pl.program_id(0)
    psum = jnp.sum(p, axis=0, keepdims=True)

    @pl.when(step == 0)
    def _():
        acc_ref[...] = psum

    @pl.when(step != 0)
    def _():
        acc_ref[...] = acc_ref[...] + psum

    @pl.when(step == _GRID - 1)
    def _():
        rp = acc_ref[...] / (_B * _S)
        aux_ref[0, 0] = jnp.sum(rp * jnp.log(rp * _E + 1e-09))


_call = pl.pallas_call(
    _router_body,
    grid=(_GRID,),
    in_specs=[
        pl.BlockSpec((1, _TS, _H), lambda i: (0, i, 0)),
        pl.BlockSpec((_H, _H), lambda i: (0, 0)),
        pl.BlockSpec((1, _H), lambda i: (0, 0)),
        pl.BlockSpec((_H, _E), lambda i: (0, 0)),
        pl.BlockSpec((1, _E), lambda i: (0, 0)),
    ],
    out_specs=[
        pl.BlockSpec((1, _TS, _E, _CAP), lambda i: (0, i, 0, 0)),
        pl.BlockSpec((1, _TS, _E, _CAP), lambda i: (0, i, 0, 0)),
        pl.BlockSpec((1, _E, _TS), lambda i: (0, 0, i)),
        pl.BlockSpec((1, 1), lambda i: (0, 0), memory_space=pltpu.SMEM),
    ],
    out_shape=[
        jax.ShapeDtypeStruct((_B, _S, _E, _CAP), jnp.float32),
        jax.ShapeDtypeStruct((_B, _S, _E, _CAP), jnp.float32),
        jax.ShapeDtypeStruct((_B, _E, _S), jnp.float32),
        jax.ShapeDtypeStruct((1, 1), jnp.float32),
    ],
    scratch_shapes=[pltpu.VMEM((1, _E), jnp.float32)],
)


def kernel(hidden_states, W1, b1, W2, b2):
    disp, comb, probs_t, aux = _call(
        hidden_states, W1, b1.reshape(1, _H), W2, b2.reshape(1, _E))
    return (disp, comb, jnp.transpose(probs_t, (0, 2, 1)), aux[0, 0])
